# Initial kernel scaffold; baseline (speedup 1.0000x reference)
import jax
import jax.numpy as jnp
from jax.experimental import pallas as pl


def kernel(hidden_states, router_weight, gate_proj, up_proj, down_proj, gate_proj_bias, up_proj_bias, down_proj_bias):
    raise NotImplementedError("write your pallas kernel here")


# speedup vs baseline: 1.2033x; 1.2033x over previous
import functools

import jax
import jax.numpy as jnp
from jax.experimental import pallas as pl
from jax.experimental.pallas import tpu as pltpu


def _sigmoid(x):
    return 1.0 / (1.0 + jnp.exp(-x))


def _router_body(x_ref, wr_ref, scores_ref):
    x = x_ref[...]
    wr = wr_ref[...]
    logits = jax.lax.dot_general(
        x, wr, (((1,), (1,)), ((), ())), preferred_element_type=jnp.float32)
    probs = _sigmoid(logits)
    n_e = probs.shape[1]
    iota = jax.lax.broadcasted_iota(jnp.int32, probs.shape, 1)
    big = jnp.int32(n_e)
    m1 = jnp.max(probs, axis=1, keepdims=True)
    i1 = jnp.min(jnp.where(probs == m1, iota, big), axis=1, keepdims=True)
    masked = jnp.where(iota == i1, -jnp.inf, probs)
    m2 = jnp.max(masked, axis=1, keepdims=True)
    i2 = jnp.min(jnp.where(masked == m2, iota, big), axis=1, keepdims=True)
    denom = m1 + m2 + 1e-20
    scores_ref[...] = jnp.where(
        iota == i1, m1 / denom, jnp.where(iota == i2, m2 / denom, 0.0))


def _moe_body(x_ref, s_ref, wg_ref, wu_ref, wd_ref, bg_ref, bu_ref, bd_ref,
              out_ref):
    e_idx = pl.program_id(1)
    x = x_ref[...]
    g = jax.lax.dot_general(
        x, wg_ref[0], (((1,), (0,)), ((), ())),
        preferred_element_type=jnp.float32) + bg_ref[0]
    u = jax.lax.dot_general(
        x, wu_ref[0], (((1,), (0,)), ((), ())),
        preferred_element_type=jnp.float32) + bu_ref[0]
    a = (g * _sigmoid(g) * u).astype(jnp.bfloat16)
    s_blk = s_ref[...]
    onehot = jax.lax.broadcasted_iota(jnp.int32, s_blk.shape, 1) == e_idx
    s_col = jnp.sum(jnp.where(onehot, s_blk, 0.0), axis=1, keepdims=True)
    contrib = (jax.lax.dot_general(
        a, wd_ref[0], (((1,), (0,)), ((), ())),
        preferred_element_type=jnp.float32) + bd_ref[0]) * s_col

    @pl.when(e_idx == 0)
    def _init():
        out_ref[...] = contrib

    @pl.when(e_idx > 0)
    def _acc():
        out_ref[...] += contrib


def kernel(hidden_states, router_weight, gate_proj, up_proj, down_proj,
           gate_proj_bias, up_proj_bias, down_proj_bias):
    b, s, h = hidden_states.shape
    e, _, i = gate_proj.shape
    t = b * s
    xf = hidden_states.reshape(t, h)
    x16 = xf.astype(jnp.bfloat16)
    wg16 = gate_proj.astype(jnp.bfloat16)
    wu16 = up_proj.astype(jnp.bfloat16)
    wd16 = down_proj.astype(jnp.bfloat16)

    rt = min(2048, t)
    scores = pl.pallas_call(
        _router_body,
        grid=(t // rt,),
        in_specs=[
            pl.BlockSpec((rt, h), lambda ti: (ti, 0)),
            pl.BlockSpec((e, h), lambda ti: (0, 0)),
        ],
        out_specs=pl.BlockSpec((rt, e), lambda ti: (ti, 0)),
        out_shape=jax.ShapeDtypeStruct((t, e), jnp.float32),
    )(xf, router_weight)

    tt = min(512, t)
    out = pl.pallas_call(
        _moe_body,
        grid=(t // tt, e),
        in_specs=[
            pl.BlockSpec((tt, h), lambda ti, ei: (ti, 0)),
            pl.BlockSpec((tt, e), lambda ti, ei: (ti, 0)),
            pl.BlockSpec((1, h, i), lambda ti, ei: (ei, 0, 0)),
            pl.BlockSpec((1, h, i), lambda ti, ei: (ei, 0, 0)),
            pl.BlockSpec((1, i, h), lambda ti, ei: (ei, 0, 0)),
            pl.BlockSpec((1, 1, i), lambda ti, ei: (ei, 0, 0)),
            pl.BlockSpec((1, 1, i), lambda ti, ei: (ei, 0, 0)),
            pl.BlockSpec((1, 1, h), lambda ti, ei: (ei, 0, 0)),
        ],
        out_specs=pl.BlockSpec((tt, h), lambda ti, ei: (ti, 0)),
        out_shape=jax.ShapeDtypeStruct((t, h), jnp.float32),
        compiler_params=pltpu.CompilerParams(
            dimension_semantics=("arbitrary", "arbitrary")),
    )(x16, scores, wg16, wu16, wd16,
      gate_proj_bias.reshape(e, 1, i), up_proj_bias.reshape(e, 1, i),
      down_proj_bias.reshape(e, 1, h))

    return out.reshape(b, s, h)


# speedup vs baseline: 1.4995x; 1.2462x over previous
import functools

import jax
import jax.numpy as jnp
from jax import lax
from jax.experimental import pallas as pl
from jax.experimental.pallas import tpu as pltpu
from jax.experimental.pallas import tpu_sc as plsc

_T, _H, _E, _I = 4096, 2048, 8, 1408
_BLK = 128
_PMAX = 2 * _T + _E * _BLK
_NB = _PMAX // _BLK
_NBP = 80
_NSUB = 16
_TCH = _T // _NSUB
_NW = 32
_LANES = 16


def _sigmoid(x):
    return 1.0 / (1.0 + jnp.exp(-x))


def _router_body(x_ref, wr_ref, i1_ref, i2_ref, w1_ref, w2_ref):
    x = x_ref[...]
    wr = wr_ref[...]
    logits = jax.lax.dot_general(
        x, wr, (((1,), (1,)), ((), ())), preferred_element_type=jnp.float32)
    probs = _sigmoid(logits)
    n_e = probs.shape[1]
    iota = lax.broadcasted_iota(jnp.int32, probs.shape, 1)
    big = jnp.int32(n_e)
    m1 = jnp.max(probs, axis=1, keepdims=True)
    i1 = jnp.min(jnp.where(probs == m1, iota, big), axis=1, keepdims=True)
    masked = jnp.where(iota == i1, -jnp.inf, probs)
    m2 = jnp.max(masked, axis=1, keepdims=True)
    i2 = jnp.min(jnp.where(masked == m2, iota, big), axis=1, keepdims=True)
    denom = m1 + m2 + 1e-20
    i1_ref[...] = i1
    i2_ref[...] = i2
    w1_ref[...] = m1 / denom
    w2_ref[...] = m2 / denom


def _sread(ref, j):
    return ref[pl.ds(j, _LANES)][0]


def _swrite(ref, j, val):
    iota = lax.broadcasted_iota(jnp.int32, (_LANES,), 0)
    v = ref[pl.ds(j, _LANES)]
    ref[pl.ds(j, _LANES)] = jnp.where(iota == 0, val, v)


def _count_body(i1_hbm, i2_hbm, cnt_hbm, i1_v, i2_v, cnt_v):
    sid = lax.axis_index("s")
    base = sid * _TCH
    iota = lax.broadcasted_iota(jnp.int32, (_LANES,), 0)
    pltpu.sync_copy(i1_hbm.at[pl.ds(base, _TCH)], i1_v.at[pl.ds(0, _TCH)])
    pltpu.sync_copy(i2_hbm.at[pl.ds(base, _TCH)], i2_v.at[pl.ds(0, _TCH)])

    def _count(j, counts):
        e1 = _sread(i1_v, j)
        counts = jnp.where(iota == e1, counts + 1, counts)
        e2 = _sread(i2_v, j)
        return jnp.where(iota == e2, counts + 1, counts)

    counts = lax.fori_loop(0, _TCH, _count, jnp.zeros((_LANES,), jnp.int32))
    cnt_v[pl.ds(0, _LANES)] = counts
    pltpu.sync_copy(cnt_v.at[pl.ds(0, _LANES)],
                    cnt_hbm.at[pl.ds(sid * _LANES, _LANES)])


def _dispatch_body(i1_hbm, i2_hbm, w1_hbm, w2_hbm, cnt_hbm,
                   rid_hbm, ws_hbm, pos1_hbm, pos2_hbm, be_hbm,
                   i1_v, i2_v, w1_v, w2_v, cnt_all, off_v,
                   pos1_v, pos2_v, be_v,
                   dpos0, dpos1, dpos2, dpos3,
                   dtok0, dtok1, dtok2, dtok3,
                   dw0, dw1, dw2, dw3):
    sid = lax.axis_index("s")
    base = sid * _TCH
    iota = lax.broadcasted_iota(jnp.int32, (_LANES,), 0)
    dpos = (dpos0, dpos1, dpos2, dpos3)
    dtok = (dtok0, dtok1, dtok2, dtok3)
    dw = (dw0, dw1, dw2, dw3)

    pltpu.sync_copy(cnt_hbm, cnt_all)
    pltpu.sync_copy(i1_hbm.at[pl.ds(base, _TCH)], i1_v.at[pl.ds(0, _TCH)])
    pltpu.sync_copy(i2_hbm.at[pl.ds(base, _TCH)], i2_v.at[pl.ds(0, _TCH)])
    pltpu.sync_copy(w1_hbm.at[pl.ds(base, _TCH)], w1_v.at[pl.ds(0, _TCH)])
    pltpu.sync_copy(w2_hbm.at[pl.ds(base, _TCH)], w2_v.at[pl.ds(0, _TCH)])

    tot, pri = [], []
    for e in range(_E):
        te = jnp.int32(0)
        pe = jnp.int32(0)
        for r in range(_NSUB):
            v = cnt_all[pl.ds(r * _LANES, _LANES)][e]
            te = te + v
            pe = pe + jnp.where(jnp.int32(r) < sid, v, jnp.int32(0))
        tot.append(te)
        pri.append(pe)
    gs = []
    acc = jnp.int32(0)
    for e in range(_E):
        gs.append(acc)
        acc = acc + ((tot[e] + _BLK - 1) // _BLK) * _BLK
    start_off = jnp.zeros((_LANES,), jnp.int32)
    for e in range(_E):
        start_off = jnp.where(iota == e, gs[e] + pri[e], start_off)
    off_v[pl.ds(0, _LANES)] = start_off

    for half, (src, wsrc, pos_v) in enumerate(
            ((i1_v, w1_v, pos1_v), (i2_v, w2_v, pos2_v))):
        for part in range(2):
            dp = dpos[half * 2 + part]
            dt = dtok[half * 2 + part]
            dwp = dw[half * 2 + part]

            def _assign(jj, carry, src=src, wsrc=wsrc, pos_v=pos_v,
                        dp=dp, dt=dt, dwp=dwp, part=part):
                j = part * 128 + jj
                e = _sread(src, j)
                p = _sread(off_v, e)
                ov = off_v[pl.ds(0, _LANES)]
                off_v[pl.ds(0, _LANES)] = jnp.where(iota == e, ov + 1, ov)
                _swrite(pos_v, j, p)
                _swrite(dp, jj, p)
                _swrite(dt, jj, base + j)
                _swrite(dwp, jj, _sread(wsrc, j))
                return carry

            lax.fori_loop(0, 128, _assign, 0)

    @pl.when(sid == 0)
    def _be():
        for bg in range(_NBP // _LANES):
            be_v[pl.ds(bg * _LANES, _LANES)] = jnp.full(
                (_LANES,), _E, jnp.int32)
        for e in range(_E):
            sb = gs[e] // _BLK
            nb = (tot[e] + _BLK - 1) // _BLK

            def _fill(bi, carry, sb=sb, e=e):
                _swrite(be_v, sb + bi, jnp.int32(e))
                return carry

            lax.fori_loop(0, nb, _fill, 0)
        pltpu.sync_copy(be_v.at[pl.ds(0, _NBP)], be_hbm)

    for r in range(4):
        pltpu.sync_copy(dtok[r].at[pl.ds(0, 128)],
                        rid_hbm.at[dpos[r].at[pl.ds(0, 128)]])
        pltpu.sync_copy(dw[r].at[pl.ds(0, 128)],
                        ws_hbm.at[dpos[r].at[pl.ds(0, 128)]])
    pltpu.sync_copy(pos1_v.at[pl.ds(0, _TCH)], pos1_hbm.at[pl.ds(base, _TCH)])
    pltpu.sync_copy(pos2_v.at[pl.ds(0, _TCH)], pos2_hbm.at[pl.ds(base, _TCH)])


def _gather_body(x_hbm, rid_hbm, xs_hbm, rid_v, buf0, buf1, sem0, sem1):
    wid = lax.axis_index("s") * 2 + lax.axis_index("c")
    rch = _PMAX // _NW
    rbase = wid * rch
    pltpu.sync_copy(rid_hbm.at[pl.ds(rbase, rch)], rid_v)
    for c in range(rch // _LANES):
        v = rid_v[pl.ds(c * _LANES, _LANES)]
        rid_v[pl.ds(c * _LANES, _LANES)] = jnp.where(
            (v >= 0) & (v < _T), v, 0)
    bufs, sems = (buf0, buf1), (sem0, sem1)
    nch = rch // _LANES
    handles = [None, None]
    handles[0] = pltpu.async_copy(
        x_hbm.at[rid_v.at[pl.ds(0, _LANES)]], bufs[0], sems[0])
    for c in range(nch):
        if c + 1 < nch:
            handles[(c + 1) % 2] = pltpu.async_copy(
                x_hbm.at[rid_v.at[pl.ds((c + 1) * _LANES, _LANES)]],
                bufs[(c + 1) % 2], sems[(c + 1) % 2])
        handles[c % 2].wait()
        pltpu.sync_copy(bufs[c % 2],
                        xs_hbm.at[pl.ds(rbase + c * _LANES, _LANES)])


def _ffn_body(be_ref, xs_ref, ws_ref, wg_ref, wu_ref, wd_ref,
              bg_ref, bu_ref, bd_ref, y_ref):
    i = pl.program_id(0)

    @pl.when(be_ref[i] < _E)
    def _():
        x = xs_ref[...].astype(jnp.bfloat16)
        g = jax.lax.dot_general(
            x, wg_ref[0], (((1,), (0,)), ((), ())),
            preferred_element_type=jnp.float32) + bg_ref[0]
        u = jax.lax.dot_general(
            x, wu_ref[0], (((1,), (0,)), ((), ())),
            preferred_element_type=jnp.float32) + bu_ref[0]
        a = (g * _sigmoid(g) * u).astype(jnp.bfloat16)
        y_ref[...] = (jax.lax.dot_general(
            a, wd_ref[0], (((1,), (0,)), ((), ())),
            preferred_element_type=jnp.float32) + bd_ref[0]) * ws_ref[...]


def _pairs_body(y_hbm, pos1_hbm, pos2_hbm, y1_hbm, y2_hbm,
                p1_v, p2_v, buf0, buf1, sem0, sem1):
    wid = lax.axis_index("s") * 2 + lax.axis_index("c")
    tw = _T // _NW
    tb = wid * tw
    pltpu.sync_copy(pos1_hbm.at[pl.ds(tb, tw)], p1_v)
    pltpu.sync_copy(pos2_hbm.at[pl.ds(tb, tw)], p2_v)
    for c in range(tw // _LANES):
        sl = pl.ds(c * _LANES, _LANES)
        osl = pl.ds(tb + c * _LANES, _LANES)
        h0 = pltpu.async_copy(y_hbm.at[p1_v.at[sl]], buf0, sem0)
        h1 = pltpu.async_copy(y_hbm.at[p2_v.at[sl]], buf1, sem1)
        h0.wait()
        pltpu.sync_copy(buf0, y1_hbm.at[osl])
        h1.wait()
        pltpu.sync_copy(buf1, y2_hbm.at[osl])


def _add_body(a_ref, b_ref, o_ref):
    o_ref[...] = a_ref[...] + b_ref[...]


def kernel(hidden_states, router_weight, gate_proj, up_proj, down_proj,
           gate_proj_bias, up_proj_bias, down_proj_bias):
    b, s, h = hidden_states.shape
    e, _, i = gate_proj.shape
    t = b * s
    xf = hidden_states.reshape(t, h)
    wg16 = gate_proj.astype(jnp.bfloat16)
    wu16 = up_proj.astype(jnp.bfloat16)
    wd16 = down_proj.astype(jnp.bfloat16)

    rt = min(2048, t)
    i1, i2, w1, w2 = pl.pallas_call(
        _router_body,
        grid=(t // rt,),
        in_specs=[
            pl.BlockSpec((rt, h), lambda ti: (ti, 0)),
            pl.BlockSpec((e, h), lambda ti: (0, 0)),
        ],
        out_specs=[
            pl.BlockSpec((rt, 1), lambda ti: (ti, 0)),
            pl.BlockSpec((rt, 1), lambda ti: (ti, 0)),
            pl.BlockSpec((rt, 1), lambda ti: (ti, 0)),
            pl.BlockSpec((rt, 1), lambda ti: (ti, 0)),
        ],
        out_shape=[
            jax.ShapeDtypeStruct((t, 1), jnp.int32),
            jax.ShapeDtypeStruct((t, 1), jnp.int32),
            jax.ShapeDtypeStruct((t, 1), jnp.float32),
            jax.ShapeDtypeStruct((t, 1), jnp.float32),
        ],
    )(xf, router_weight)
    i1, i2 = i1.reshape(t), i2.reshape(t)
    w1, w2 = w1.reshape(t), w2.reshape(t)

    mesh1 = plsc.VectorSubcoreMesh(
        core_axis_name="c", subcore_axis_name="s", num_cores=1)
    count_k = functools.partial(
        pl.kernel, mesh=mesh1,
        out_type=jax.ShapeDtypeStruct((_NSUB * _LANES,), jnp.int32),
        scratch_types=[
            pltpu.VMEM((_TCH + 16,), jnp.int32),
            pltpu.VMEM((_TCH + 16,), jnp.int32),
            pltpu.VMEM((_LANES + 16,), jnp.int32),
        ])(_count_body)
    cnt = count_k(i1, i2)

    dispatch = functools.partial(
        pl.kernel, mesh=mesh1,
        out_type=[
            jax.ShapeDtypeStruct((_PMAX,), jnp.int32),
            jax.ShapeDtypeStruct((_PMAX,), jnp.float32),
            jax.ShapeDtypeStruct((_T,), jnp.int32),
            jax.ShapeDtypeStruct((_T,), jnp.int32),
            jax.ShapeDtypeStruct((_NBP,), jnp.int32),
        ],
        scratch_types=[
            pltpu.VMEM((_TCH + 16,), jnp.int32),
            pltpu.VMEM((_TCH + 16,), jnp.int32),
            pltpu.VMEM((_TCH + 16,), jnp.float32),
            pltpu.VMEM((_TCH + 16,), jnp.float32),
            pltpu.VMEM((_NSUB * _LANES,), jnp.int32),
            pltpu.VMEM((_LANES + 16,), jnp.int32),
            pltpu.VMEM((_TCH + 16,), jnp.int32),
            pltpu.VMEM((_TCH + 16,), jnp.int32),
            pltpu.VMEM((_NBP + 16,), jnp.int32),
            pltpu.VMEM((144,), jnp.int32),
            pltpu.VMEM((144,), jnp.int32),
            pltpu.VMEM((144,), jnp.int32),
            pltpu.VMEM((144,), jnp.int32),
            pltpu.VMEM((144,), jnp.int32),
            pltpu.VMEM((144,), jnp.int32),
            pltpu.VMEM((144,), jnp.int32),
            pltpu.VMEM((144,), jnp.int32),
            pltpu.VMEM((144,), jnp.float32),
            pltpu.VMEM((144,), jnp.float32),
            pltpu.VMEM((144,), jnp.float32),
            pltpu.VMEM((144,), jnp.float32),
        ])(_dispatch_body)
    row_id, w_sorted, pos1, pos2, block_expert = dispatch(i1, i2, w1, w2, cnt)


    mesh2 = plsc.VectorSubcoreMesh(core_axis_name="c", subcore_axis_name="s")
    gather = functools.partial(
        pl.kernel, mesh=mesh2,
        out_type=jax.ShapeDtypeStruct((_PMAX, h), jnp.float32),
        scratch_types=[
            pltpu.VMEM((_PMAX // _NW,), jnp.int32),
            pltpu.VMEM((_LANES, h), jnp.float32),
            pltpu.VMEM((_LANES, h), jnp.float32),
            pltpu.SemaphoreType.DMA,
            pltpu.SemaphoreType.DMA,
        ])(_gather_body)
    xs = gather(xf, row_id)

    y = pl.pallas_call(
        _ffn_body,
        grid_spec=pltpu.PrefetchScalarGridSpec(
            num_scalar_prefetch=1,
            grid=(_NB,),
            in_specs=[
                pl.BlockSpec((_BLK, h), lambda ib, be: (ib, 0)),
                pl.BlockSpec((_BLK, 1), lambda ib, be: (ib, 0)),
                pl.BlockSpec((1, h, i),
                             lambda ib, be: (jnp.minimum(be[ib], e - 1), 0, 0)),
                pl.BlockSpec((1, h, i),
                             lambda ib, be: (jnp.minimum(be[ib], e - 1), 0, 0)),
                pl.BlockSpec((1, i, h),
                             lambda ib, be: (jnp.minimum(be[ib], e - 1), 0, 0)),
                pl.BlockSpec((1, 1, i),
                             lambda ib, be: (jnp.minimum(be[ib], e - 1), 0, 0)),
                pl.BlockSpec((1, 1, i),
                             lambda ib, be: (jnp.minimum(be[ib], e - 1), 0, 0)),
                pl.BlockSpec((1, 1, h),
                             lambda ib, be: (jnp.minimum(be[ib], e - 1), 0, 0)),
            ],
            out_specs=pl.BlockSpec((_BLK, h), lambda ib, be: (ib, 0)),
        ),
        out_shape=jax.ShapeDtypeStruct((_PMAX, h), jnp.float32),
        compiler_params=pltpu.CompilerParams(
            dimension_semantics=("arbitrary",)),
    )(block_expert, xs, w_sorted.reshape(_PMAX, 1), wg16, wu16, wd16,
      gate_proj_bias.reshape(e, 1, i), up_proj_bias.reshape(e, 1, i),
      down_proj_bias.reshape(e, 1, h))

    pairs = functools.partial(
        pl.kernel, mesh=mesh2,
        out_type=[
            jax.ShapeDtypeStruct((t, h), jnp.float32),
            jax.ShapeDtypeStruct((t, h), jnp.float32),
        ],
        scratch_types=[
            pltpu.VMEM((_T // _NW,), jnp.int32),
            pltpu.VMEM((_T // _NW,), jnp.int32),
            pltpu.VMEM((_LANES, h), jnp.float32),
            pltpu.VMEM((_LANES, h), jnp.float32),
            pltpu.SemaphoreType.DMA,
            pltpu.SemaphoreType.DMA,
        ])(_pairs_body)
    y1g, y2g = pairs(y, pos1, pos2)

    at = min(512, t)
    out = pl.pallas_call(
        _add_body,
        grid=(t // at,),
        in_specs=[
            pl.BlockSpec((at, h), lambda ti: (ti, 0)),
            pl.BlockSpec((at, h), lambda ti: (ti, 0)),
        ],
        out_specs=pl.BlockSpec((at, h), lambda ti: (ti, 0)),
        out_shape=jax.ShapeDtypeStruct((t, h), jnp.float32),
    )(y1g, y2g)
    return out.reshape(b, s, h)
